# Initial kernel scaffold; baseline (speedup 1.0000x reference)
import jax
import jax.numpy as jnp
from jax.experimental import pallas as pl


def kernel(h_pooled, W1, b1, W2, b2):
    raise NotImplementedError("write your pallas kernel here")


# speedup vs baseline: 1.6178x; 1.6178x over previous
import functools

import jax
import jax.numpy as jnp
from jax.experimental import pallas as pl

B = 32768
D_TEA = 768
GATE_H = 512
NUM_EXPERTS = 64
TOP_K = 8
BLK = 1024


def _router_block(h_ref, w1_ref, b1_ref, w2_ref, b2_ref,
                  tkw_ref, tki_ref, probs_ref):
    h = h_ref[...]
    hidden = jnp.dot(h, w1_ref[...], preferred_element_type=jnp.float32)
    hidden = hidden + b1_ref[...]
    hidden = 0.5 * hidden * (1.0 + jax.lax.erf(hidden * (2.0 ** -0.5)))
    logits = jnp.dot(hidden, w2_ref[...], preferred_element_type=jnp.float32)
    logits = logits + b2_ref[...]

    m = jnp.max(logits, axis=-1, keepdims=True)
    e = jnp.exp(logits - m)
    probs = e / jnp.sum(e, axis=-1, keepdims=True)
    probs_ref[...] = probs

    idx = jax.lax.broadcasted_iota(jnp.int32, probs.shape, 1)
    work = probs
    vals = []
    inds = []
    for _ in range(TOP_K):
        mx = jnp.max(work, axis=-1, keepdims=True)
        am = jnp.min(jnp.where(work == mx, idx, NUM_EXPERTS), axis=-1,
                     keepdims=True)
        vals.append(mx)
        inds.append(am)
        work = jnp.where(idx == am, -1.0, work)
    tkw = jnp.concatenate(vals, axis=-1)
    tki = jnp.concatenate(inds, axis=-1)
    tkw_ref[...] = tkw / (jnp.sum(tkw, axis=-1, keepdims=True) + 1e-08)
    tki_ref[...] = tki


@functools.partial(jax.jit, donate_argnums=())
def kernel(h_pooled, W1, b1, W2, b2):
    grid = (B // BLK,)
    out = pl.pallas_call(
        _router_block,
        grid=grid,
        in_specs=[
            pl.BlockSpec((BLK, D_TEA), lambda i: (i, 0)),
            pl.BlockSpec((D_TEA, GATE_H), lambda i: (0, 0)),
            pl.BlockSpec((GATE_H,), lambda i: (0,)),
            pl.BlockSpec((GATE_H, NUM_EXPERTS), lambda i: (0, 0)),
            pl.BlockSpec((NUM_EXPERTS,), lambda i: (0,)),
        ],
        out_specs=[
            pl.BlockSpec((BLK, TOP_K), lambda i: (i, 0)),
            pl.BlockSpec((BLK, TOP_K), lambda i: (i, 0)),
            pl.BlockSpec((BLK, NUM_EXPERTS), lambda i: (i, 0)),
        ],
        out_shape=[
            jax.ShapeDtypeStruct((B, TOP_K), jnp.float32),
            jax.ShapeDtypeStruct((B, TOP_K), jnp.int32),
            jax.ShapeDtypeStruct((B, NUM_EXPERTS), jnp.float32),
        ],
    )(h_pooled, W1, b1, W2, b2)
    return tuple(out)
